# Initial kernel scaffold; baseline (speedup 1.0000x reference)
import jax
import jax.numpy as jnp
from jax.experimental import pallas as pl


def kernel(edge_index, edge_vals, user_preference, item_preference, user_map, item_map, U_mul_S, V_mul_S):
    raise NotImplementedError("write your pallas kernel here")


# speedup vs baseline: 3.0293x; 3.0293x over previous
import functools

import jax
import jax.numpy as jnp
from jax import lax
from jax.experimental import pallas as pl
from jax.experimental.pallas import tpu as pltpu
from jax.experimental.pallas import tpu_sc as plsc

N_NODES = 25000
D = 64
E = 800000
CHUNK = 128
N_CHUNKS = E // CHUNK
ITERS = (N_CHUNKS + 15) // 16
ZROWS = 128
NZFULL = N_NODES // ZROWS
ZREM = N_NODES - NZFULL * ZROWS
ZITERS = (NZFULL + 15) // 16


def _propagate_body(rows_hbm, cols_hbm, vals_hbm, u_hbm, i_hbm,
                    new_u, new_i, acc, didx, sidx, vbuf, rowbuf, sem):
    sid = lax.axis_index("s")
    core = lax.axis_index("c")

    def run_direction(dst_hbm, src_hbm, table_hbm, out_hbm):
        def zero_rowbuf(r, c):
            for j in range(4):
                rowbuf[r, pl.ds(j * 16, 16)] = jnp.zeros((16,), jnp.float32)
            return c
        lax.fori_loop(0, ZROWS, zero_rowbuf, 0)

        def zero_acc(it, c):
            cid = it * 16 + sid

            @pl.when(cid < NZFULL)
            def _():
                pltpu.sync_copy(rowbuf, acc.at[pl.ds(cid * ZROWS, ZROWS)])
            return c
        lax.fori_loop(0, ZITERS, zero_acc, 0)

        @pl.when(sid == 0)
        def _():
            pltpu.sync_copy(rowbuf.at[pl.ds(0, ZREM)],
                            acc.at[pl.ds(NZFULL * ZROWS, ZREM)])

        plsc.subcore_barrier()

        def edge_chunk(it, c):
            cid = it * 16 + sid

            @pl.when(cid < N_CHUNKS)
            def _():
                base = cid * CHUNK
                pltpu.sync_copy(dst_hbm.at[pl.ds(base, CHUNK)], didx)
                pltpu.sync_copy(src_hbm.at[pl.ds(base, CHUNK)], sidx)
                pltpu.sync_copy(vals_hbm.at[pl.ds(base, CHUNK)], vbuf)
                pltpu.async_copy(table_hbm.at[sidx], rowbuf, sem).wait()

                def scale(g, c2):
                    vv = vbuf[pl.ds(g * 16, 16)]
                    for l in range(16):
                        k = g * 16 + l
                        v = vv[l]
                        for j in range(4):
                            sl = pl.ds(j * 16, 16)
                            rowbuf[k, sl] = rowbuf[k, sl] * v
                    return c2
                lax.fori_loop(0, CHUNK // 16, scale, 0)

                pltpu.sync_copy(rowbuf, acc.at[didx], add=True)
            return c
        lax.fori_loop(0, ITERS, edge_chunk, 0)

        plsc.subcore_barrier()

        def export(it, c):
            cid = it * 16 + sid

            @pl.when(cid < NZFULL)
            def _():
                sl = pl.ds(cid * ZROWS, ZROWS)
                pltpu.sync_copy(acc.at[sl], out_hbm.at[sl])
            return c
        lax.fori_loop(0, ZITERS, export, 0)

        @pl.when(sid == 0)
        def _():
            sl = pl.ds(NZFULL * ZROWS, ZREM)
            pltpu.sync_copy(acc.at[sl], out_hbm.at[sl])

    @pl.when(core == 0)
    def _():
        run_direction(rows_hbm, cols_hbm, i_hbm, new_u)

    @pl.when(core == 1)
    def _():
        run_direction(cols_hbm, rows_hbm, u_hbm, new_i)


_propagate = functools.partial(
    pl.kernel,
    out_type=(jax.ShapeDtypeStruct((N_NODES, D), jnp.float32),
              jax.ShapeDtypeStruct((N_NODES, D), jnp.float32)),
    mesh=plsc.VectorSubcoreMesh(core_axis_name="c", subcore_axis_name="s"),
    scratch_types=[
        pltpu.VMEM_SHARED((N_NODES, D), jnp.float32),
        pltpu.VMEM((CHUNK,), jnp.int32),
        pltpu.VMEM((CHUNK,), jnp.int32),
        pltpu.VMEM((CHUNK,), jnp.float32),
        pltpu.VMEM((CHUNK, D), jnp.float32),
        pltpu.SemaphoreType.DMA,
    ],
    compiler_params=pltpu.CompilerParams(use_tc_tiling_on_sc=False),
)(_propagate_body)


ROWS_BLK = 1000


def _final_body(u0, u1, u2, u3, i0, i1, i2, i3, us, vs, umap, imap, out):
    out[0, :, :] = (u0[...] + u1[...] + u2[...] + u3[...]) * 0.25
    out[1, :, :] = (i0[...] + i1[...] + i2[...] + i3[...]) * 0.25
    out[2, :, :] = jnp.dot(us[...], umap[...],
                           preferred_element_type=jnp.float32)
    out[3, :, :] = jnp.dot(vs[...], imap[...],
                           preferred_element_type=jnp.float32)


def _finalize(u0, u1, u2, u3, i0, i1, i2, i3, us, vs, umap, imap):
    row_spec = pl.BlockSpec((ROWS_BLK, D), lambda i: (i, 0))
    map_spec = pl.BlockSpec((D, D), lambda i: (0, 0))
    return pl.pallas_call(
        _final_body,
        grid=(N_NODES // ROWS_BLK,),
        in_specs=[row_spec] * 10 + [map_spec] * 2,
        out_specs=pl.BlockSpec((4, ROWS_BLK, D), lambda i: (0, i, 0)),
        out_shape=jax.ShapeDtypeStruct((4, N_NODES, D), jnp.float32),
    )(u0, u1, u2, u3, i0, i1, i2, i3, us, vs, umap, imap)


def kernel(edge_index, edge_vals, user_preference, item_preference,
           user_map, item_map, U_mul_S, V_mul_S):
    rows = edge_index[0].astype(jnp.int32)
    cols = edge_index[1].astype(jnp.int32)
    vals = edge_vals.astype(jnp.float32)

    u0, i0 = user_preference, item_preference
    u1, i1 = _propagate(rows, cols, vals, u0, i0)
    u2, i2 = _propagate(rows, cols, vals, u1, i1)
    u3, i3 = _propagate(rows, cols, vals, u2, i2)

    return _finalize(u0, u1, u2, u3, i0, i1, i2, i3,
                     U_mul_S, V_mul_S, user_map, item_map)


# speedup vs baseline: 4.2549x; 1.4046x over previous
import functools

import jax
import jax.numpy as jnp
from jax import lax
from jax.experimental import pallas as pl
from jax.experimental.pallas import tpu as pltpu
from jax.experimental.pallas import tpu_sc as plsc

N_NODES = 25000
D = 64
E = 800000
CHUNK = 128
CPT = 391
E_PAD = 16 * CPT * CHUNK
ROWS2 = E_PAD // CHUNK
BANK = 17
NBANKS = CPT // BANK
NBUF = 2
G = 1
ZROWS = 128
NZFULL = N_NODES // ZROWS
ZREM = N_NODES - NZFULL * ZROWS
ZITERS = (NZFULL + 15) // 16


def _propagate_body(rows_hbm, cols_hbm, vals_hbm, u_hbm, i_hbm,
                    new_u, new_i, acc, didx, sidx, vbank, rowbufs,
                    gsem, ssem, isem):
    sid = lax.axis_index("s")
    core = lax.axis_index("c")

    def run_direction(dst_hbm, src_hbm, table_hbm, out_hbm):
        base_row = sid * CPT

        def zero_rowbuf(r, c):
            for j in range(4):
                rowbufs[0, r, pl.ds(j * 16, 16)] = jnp.zeros((16,),
                                                             jnp.float32)
            return c
        lax.fori_loop(0, ZROWS, zero_rowbuf, 0)

        def zero_acc(it, c):
            cid = it * 16 + sid

            @pl.when(cid < NZFULL)
            def _():
                pltpu.sync_copy(rowbufs.at[0],
                                acc.at[pl.ds(cid * ZROWS, ZROWS)])
            return c
        lax.fori_loop(0, ZITERS, zero_acc, 0)

        @pl.when(sid == 0)
        def _():
            pltpu.sync_copy(rowbufs.at[0].at[pl.ds(0, ZREM)],
                            acc.at[pl.ds(NZFULL * ZROWS, ZREM)])

        plsc.subcore_barrier()

        pltpu.sync_copy(dst_hbm.at[pl.ds(base_row, BANK)], didx.at[0])
        pltpu.sync_copy(src_hbm.at[pl.ds(base_row, BANK)], sidx.at[0])
        pltpu.sync_copy(vals_hbm.at[pl.ds(base_row, BANK)], vbank.at[0])
        for pj in range(G):
            pltpu.async_copy(table_hbm.at[sidx.at[0, pj]],
                             rowbufs.at[pj], gsem)

        def edge_chunk(j, carry):
            jb, b, rg, bg = carry
            p = b & 1
            pg = bg & 1
            slot = j & (NBUF - 1)

            @pl.when(jnp.logical_and(jb == BANK - G, b < NBANKS - 1))
            def _():
                for _k in range(3):
                    pltpu.make_async_copy(
                        dst_hbm.at[pl.ds(base_row, BANK)],
                        didx.at[1 - p], isem).wait()

            pltpu.make_async_copy(table_hbm.at[sidx.at[p, jb]],
                                  rowbufs.at[slot], gsem).wait()

            def scale(g2, c2):
                vv = vbank[p, jb, pl.ds(g2 * 16, 16)]
                for l in range(16):
                    k = g2 * 16 + l
                    v = vv[l]
                    for jj in range(4):
                        sl = pl.ds(jj * 16, 16)
                        rowbufs[slot, k, sl] = rowbufs[slot, k, sl] * v
                return c2
            lax.fori_loop(0, CHUNK // 16, scale, 0, unroll=2)

            pltpu.async_copy(rowbufs.at[slot], acc.at[didx.at[p, jb]],
                             ssem, add=True)

            @pl.when(j >= NBUF - G)
            def _():
                pltpu.make_async_copy(rowbufs.at[0], acc.at[didx.at[0, 0]],
                                      ssem).wait()

            @pl.when(j + G < CPT)
            def _():
                pltpu.async_copy(table_hbm.at[sidx.at[pg, rg]],
                                 rowbufs.at[(j + G) & (NBUF - 1)], gsem)

            @pl.when(jnp.logical_and(jb == 1, b < NBANKS - 1))
            def _():
                off = base_row + (b + 1) * BANK
                pltpu.async_copy(dst_hbm.at[pl.ds(off, BANK)],
                                 didx.at[1 - p], isem)
                pltpu.async_copy(src_hbm.at[pl.ds(off, BANK)],
                                 sidx.at[1 - p], isem)
                pltpu.async_copy(vals_hbm.at[pl.ds(off, BANK)],
                                 vbank.at[1 - p], isem)

            jb = jb + 1
            wrap = jb == BANK
            b = jnp.where(wrap, b + 1, b)
            jb = jnp.where(wrap, 0, jb)
            rg = rg + 1
            wrapg = rg == BANK
            bg = jnp.where(wrapg, bg + 1, bg)
            rg = jnp.where(wrapg, 0, rg)
            return (jb, b, rg, bg)

        lax.fori_loop(0, CPT, edge_chunk,
                      (jnp.int32(0), jnp.int32(0),
                       jnp.int32(G), jnp.int32(0)))

        for _k in range(NBUF - G):
            pltpu.make_async_copy(rowbufs.at[0], acc.at[didx.at[0, 0]],
                                  ssem).wait()

        plsc.subcore_barrier()

        def export(it, c):
            cid = it * 16 + sid

            @pl.when(cid < NZFULL)
            def _():
                sl = pl.ds(cid * ZROWS, ZROWS)
                pltpu.sync_copy(acc.at[sl], out_hbm.at[sl])
            return c
        lax.fori_loop(0, ZITERS, export, 0)

        @pl.when(sid == 0)
        def _():
            sl = pl.ds(NZFULL * ZROWS, ZREM)
            pltpu.sync_copy(acc.at[sl], out_hbm.at[sl])

    @pl.when(core == 0)
    def _():
        run_direction(rows_hbm, cols_hbm, i_hbm, new_u)

    @pl.when(core == 1)
    def _():
        run_direction(cols_hbm, rows_hbm, u_hbm, new_i)


_propagate = functools.partial(
    pl.kernel,
    out_type=(jax.ShapeDtypeStruct((N_NODES, D), jnp.float32),
              jax.ShapeDtypeStruct((N_NODES, D), jnp.float32)),
    mesh=plsc.VectorSubcoreMesh(core_axis_name="c", subcore_axis_name="s"),
    scratch_types=[
        pltpu.VMEM_SHARED((N_NODES, D), jnp.float32),
        pltpu.VMEM((2, BANK, CHUNK), jnp.int32),
        pltpu.VMEM((2, BANK, CHUNK), jnp.int32),
        pltpu.VMEM((2, BANK, CHUNK), jnp.float32),
        pltpu.VMEM((NBUF, CHUNK, D), jnp.float32),
        pltpu.SemaphoreType.DMA,
        pltpu.SemaphoreType.DMA,
        pltpu.SemaphoreType.DMA,
    ],
    compiler_params=pltpu.CompilerParams(use_tc_tiling_on_sc=False),
)(_propagate_body)


ROWS_BLK = 1000


def _final_body(u0, u1, u2, u3, i0, i1, i2, i3, us, vs, umap, imap, out):
    out[0, :, :] = (u0[...] + u1[...] + u2[...] + u3[...]) * 0.25
    out[1, :, :] = (i0[...] + i1[...] + i2[...] + i3[...]) * 0.25
    out[2, :, :] = jnp.dot(us[...], umap[...],
                           preferred_element_type=jnp.float32)
    out[3, :, :] = jnp.dot(vs[...], imap[...],
                           preferred_element_type=jnp.float32)


def _finalize(u0, u1, u2, u3, i0, i1, i2, i3, us, vs, umap, imap):
    row_spec = pl.BlockSpec((ROWS_BLK, D), lambda i: (i, 0))
    map_spec = pl.BlockSpec((D, D), lambda i: (0, 0))
    return pl.pallas_call(
        _final_body,
        grid=(N_NODES // ROWS_BLK,),
        in_specs=[row_spec] * 10 + [map_spec] * 2,
        out_specs=pl.BlockSpec((4, ROWS_BLK, D), lambda i: (0, i, 0)),
        out_shape=jax.ShapeDtypeStruct((4, N_NODES, D), jnp.float32),
    )(u0, u1, u2, u3, i0, i1, i2, i3, us, vs, umap, imap)


def kernel(edge_index, edge_vals, user_preference, item_preference,
           user_map, item_map, U_mul_S, V_mul_S):
    rows = edge_index[0].astype(jnp.int32)
    cols = edge_index[1].astype(jnp.int32)
    vals = edge_vals.astype(jnp.float32)

    pad = E_PAD - E
    rows2 = jnp.concatenate(
        [rows, jnp.zeros((pad,), jnp.int32)]).reshape(ROWS2, CHUNK)
    cols2 = jnp.concatenate(
        [cols, jnp.zeros((pad,), jnp.int32)]).reshape(ROWS2, CHUNK)
    vals2 = jnp.concatenate(
        [vals, jnp.zeros((pad,), jnp.float32)]).reshape(ROWS2, CHUNK)

    u0, i0 = user_preference, item_preference
    u1, i1 = _propagate(rows2, cols2, vals2, u0, i0)
    u2, i2 = _propagate(rows2, cols2, vals2, u1, i1)
    u3, i3 = _propagate(rows2, cols2, vals2, u2, i2)

    return _finalize(u0, u1, u2, u3, i0, i1, i2, i3,
                     U_mul_S, V_mul_S, user_map, item_map)


# speedup vs baseline: 5.6186x; 1.3205x over previous
import functools

import jax
import jax.numpy as jnp
from jax import lax
from jax.experimental import pallas as pl
from jax.experimental.pallas import tpu as pltpu
from jax.experimental.pallas import tpu_sc as plsc

N_NODES = 25000
D = 64
E = 800000
CHUNK = 64
CPT = 782
E_PAD = 16 * CPT * CHUNK
ROWS2 = E_PAD // CHUNK
BANK = 23
NBANKS = CPT // BANK
NBUF = 4
G = 2
ZROWS = 64
NZFULL = N_NODES // ZROWS
ZREM = N_NODES - NZFULL * ZROWS
ZITERS = (NZFULL + 15) // 16


def _propagate_body(rows_hbm, cols_hbm, vals_hbm, u_hbm, i_hbm,
                    new_u, new_i, acc, didx, sidx, vbank, rowbufs,
                    gsem, ssem, isem):
    sid = lax.axis_index("s")
    core = lax.axis_index("c")

    def run_direction(dst_hbm, src_hbm, table_hbm, out_hbm):
        base_row = sid * CPT

        def zero_rowbuf(r, c):
            for j in range(4):
                rowbufs[0, r, pl.ds(j * 16, 16)] = jnp.zeros((16,),
                                                             jnp.float32)
            return c
        lax.fori_loop(0, ZROWS, zero_rowbuf, 0)

        def zero_acc(it, c):
            cid = it * 16 + sid

            @pl.when(cid < NZFULL)
            def _():
                pltpu.sync_copy(rowbufs.at[0],
                                acc.at[pl.ds(cid * ZROWS, ZROWS)])
            return c
        lax.fori_loop(0, ZITERS, zero_acc, 0)

        @pl.when(sid == 0)
        def _():
            pltpu.sync_copy(rowbufs.at[0].at[pl.ds(0, ZREM)],
                            acc.at[pl.ds(NZFULL * ZROWS, ZREM)])

        plsc.subcore_barrier()

        pltpu.sync_copy(dst_hbm.at[pl.ds(base_row, BANK)], didx.at[0])
        pltpu.sync_copy(src_hbm.at[pl.ds(base_row, BANK)], sidx.at[0])
        pltpu.sync_copy(vals_hbm.at[pl.ds(base_row, BANK)], vbank.at[0])
        for pj in range(G):
            pltpu.async_copy(table_hbm.at[sidx.at[0, pj]],
                             rowbufs.at[pj], gsem)

        def edge_chunk(j, carry):
            jb, b, rg, bg = carry
            p = b & 1
            pg = bg & 1
            slot = j & (NBUF - 1)

            @pl.when(jnp.logical_and(jb == BANK - G, b < NBANKS - 1))
            def _():
                for _k in range(3):
                    pltpu.make_async_copy(
                        dst_hbm.at[pl.ds(base_row, BANK)],
                        didx.at[1 - p], isem).wait()

            pltpu.make_async_copy(table_hbm.at[sidx.at[p, jb]],
                                  rowbufs.at[slot], gsem).wait()

            def scale(g2, c2):
                vv = vbank[p, jb, pl.ds(g2 * 16, 16)]
                for l in range(16):
                    k = g2 * 16 + l
                    v = vv[l]
                    for jj in range(4):
                        sl = pl.ds(jj * 16, 16)
                        rowbufs[slot, k, sl] = rowbufs[slot, k, sl] * v
                return c2
            lax.fori_loop(0, CHUNK // 16, scale, 0, unroll=2)

            pltpu.async_copy(rowbufs.at[slot], acc.at[didx.at[p, jb]],
                             ssem, add=True)

            @pl.when(j >= NBUF - G)
            def _():
                pltpu.make_async_copy(rowbufs.at[0], acc.at[didx.at[0, 0]],
                                      ssem).wait()

            @pl.when(j + G < CPT)
            def _():
                pltpu.async_copy(table_hbm.at[sidx.at[pg, rg]],
                                 rowbufs.at[(j + G) & (NBUF - 1)], gsem)

            @pl.when(jnp.logical_and(jb == 1, b < NBANKS - 1))
            def _():
                off = base_row + (b + 1) * BANK
                pltpu.async_copy(dst_hbm.at[pl.ds(off, BANK)],
                                 didx.at[1 - p], isem)
                pltpu.async_copy(src_hbm.at[pl.ds(off, BANK)],
                                 sidx.at[1 - p], isem)
                pltpu.async_copy(vals_hbm.at[pl.ds(off, BANK)],
                                 vbank.at[1 - p], isem)

            jb = jb + 1
            wrap = jb == BANK
            b = jnp.where(wrap, b + 1, b)
            jb = jnp.where(wrap, 0, jb)
            rg = rg + 1
            wrapg = rg == BANK
            bg = jnp.where(wrapg, bg + 1, bg)
            rg = jnp.where(wrapg, 0, rg)
            return (jb, b, rg, bg)

        lax.fori_loop(0, CPT, edge_chunk,
                      (jnp.int32(0), jnp.int32(0),
                       jnp.int32(G), jnp.int32(0)))

        for _k in range(NBUF - G):
            pltpu.make_async_copy(rowbufs.at[0], acc.at[didx.at[0, 0]],
                                  ssem).wait()

        plsc.subcore_barrier()

        def export(it, c):
            cid = it * 16 + sid

            @pl.when(cid < NZFULL)
            def _():
                sl = pl.ds(cid * ZROWS, ZROWS)
                pltpu.sync_copy(acc.at[sl], out_hbm.at[sl])
            return c
        lax.fori_loop(0, ZITERS, export, 0)

        @pl.when(sid == 0)
        def _():
            sl = pl.ds(NZFULL * ZROWS, ZREM)
            pltpu.sync_copy(acc.at[sl], out_hbm.at[sl])

    @pl.when(core == 0)
    def _():
        run_direction(rows_hbm, cols_hbm, i_hbm, new_u)

    @pl.when(core == 1)
    def _():
        run_direction(cols_hbm, rows_hbm, u_hbm, new_i)


_propagate = functools.partial(
    pl.kernel,
    out_type=(jax.ShapeDtypeStruct((N_NODES, D), jnp.float32),
              jax.ShapeDtypeStruct((N_NODES, D), jnp.float32)),
    mesh=plsc.VectorSubcoreMesh(core_axis_name="c", subcore_axis_name="s"),
    scratch_types=[
        pltpu.VMEM_SHARED((N_NODES, D), jnp.float32),
        pltpu.VMEM((2, BANK, CHUNK), jnp.int32),
        pltpu.VMEM((2, BANK, CHUNK), jnp.int32),
        pltpu.VMEM((2, BANK, CHUNK), jnp.float32),
        pltpu.VMEM((NBUF, CHUNK, D), jnp.float32),
        pltpu.SemaphoreType.DMA,
        pltpu.SemaphoreType.DMA,
        pltpu.SemaphoreType.DMA,
    ],
    compiler_params=pltpu.CompilerParams(use_tc_tiling_on_sc=False),
)(_propagate_body)


ROWS_BLK = 1000


def _final_body(u0, u1, u2, u3, i0, i1, i2, i3, us, vs, umap, imap, out):
    out[0, :, :] = (u0[...] + u1[...] + u2[...] + u3[...]) * 0.25
    out[1, :, :] = (i0[...] + i1[...] + i2[...] + i3[...]) * 0.25
    out[2, :, :] = jnp.dot(us[...], umap[...],
                           preferred_element_type=jnp.float32)
    out[3, :, :] = jnp.dot(vs[...], imap[...],
                           preferred_element_type=jnp.float32)


def _finalize(u0, u1, u2, u3, i0, i1, i2, i3, us, vs, umap, imap):
    row_spec = pl.BlockSpec((ROWS_BLK, D), lambda i: (i, 0))
    map_spec = pl.BlockSpec((D, D), lambda i: (0, 0))
    return pl.pallas_call(
        _final_body,
        grid=(N_NODES // ROWS_BLK,),
        in_specs=[row_spec] * 10 + [map_spec] * 2,
        out_specs=pl.BlockSpec((4, ROWS_BLK, D), lambda i: (0, i, 0)),
        out_shape=jax.ShapeDtypeStruct((4, N_NODES, D), jnp.float32),
    )(u0, u1, u2, u3, i0, i1, i2, i3, us, vs, umap, imap)


def kernel(edge_index, edge_vals, user_preference, item_preference,
           user_map, item_map, U_mul_S, V_mul_S):
    rows = edge_index[0].astype(jnp.int32)
    cols = edge_index[1].astype(jnp.int32)
    vals = edge_vals.astype(jnp.float32)

    pad = E_PAD - E
    rows2 = jnp.concatenate(
        [rows, jnp.zeros((pad,), jnp.int32)]).reshape(ROWS2, CHUNK)
    cols2 = jnp.concatenate(
        [cols, jnp.zeros((pad,), jnp.int32)]).reshape(ROWS2, CHUNK)
    vals2 = jnp.concatenate(
        [vals, jnp.zeros((pad,), jnp.float32)]).reshape(ROWS2, CHUNK)

    u0, i0 = user_preference, item_preference
    u1, i1 = _propagate(rows2, cols2, vals2, u0, i0)
    u2, i2 = _propagate(rows2, cols2, vals2, u1, i1)
    u3, i3 = _propagate(rows2, cols2, vals2, u2, i2)

    return _finalize(u0, u1, u2, u3, i0, i1, i2, i3,
                     U_mul_S, V_mul_S, user_map, item_map)


# speedup vs baseline: 5.6216x; 1.0005x over previous
import functools

import jax
import jax.numpy as jnp
from jax import lax
from jax.experimental import pallas as pl
from jax.experimental.pallas import tpu as pltpu
from jax.experimental.pallas import tpu_sc as plsc

N_NODES = 25000
D = 64
E = 800000
CHUNK = 64
CPT = 782
E_PAD = 16 * CPT * CHUNK
ROWS2 = E_PAD // CHUNK
BANK = 23
NBANKS = CPT // BANK
NBUF = 4
G = 2
ZROWS = 64
NZFULL = N_NODES // ZROWS
ZREM = N_NODES - NZFULL * ZROWS
ZITERS = (NZFULL + 15) // 16


def _propagate_body(rows_hbm, cols_hbm, vals_hbm, u_hbm, i_hbm,
                    new_u, new_i, acc, didx, sidx, vbank, rowbufs,
                    gsem, ssem, isem):
    sid = lax.axis_index("s")
    core = lax.axis_index("c")

    def run_direction(dst_hbm, src_hbm, table_hbm, out_hbm):
        base_row = sid * CPT

        def zero_rowbuf(r, c):
            for j in range(4):
                rowbufs[0, r, pl.ds(j * 16, 16)] = jnp.zeros((16,),
                                                             jnp.float32)
            return c
        lax.fori_loop(0, ZROWS, zero_rowbuf, 0)

        def zero_acc(it, c):
            cid = it * 16 + sid

            @pl.when(cid < NZFULL)
            def _():
                pltpu.sync_copy(rowbufs.at[0],
                                acc.at[pl.ds(cid * ZROWS, ZROWS)])
            return c
        lax.fori_loop(0, ZITERS, zero_acc, 0)

        @pl.when(sid == 0)
        def _():
            pltpu.sync_copy(rowbufs.at[0].at[pl.ds(0, ZREM)],
                            acc.at[pl.ds(NZFULL * ZROWS, ZREM)])

        plsc.subcore_barrier()

        pltpu.sync_copy(dst_hbm.at[pl.ds(base_row, BANK)], didx.at[0])
        pltpu.sync_copy(src_hbm.at[pl.ds(base_row, BANK)], sidx.at[0])
        pltpu.sync_copy(vals_hbm.at[pl.ds(base_row, BANK)], vbank.at[0])
        for pj in range(G):
            pltpu.async_copy(table_hbm.at[sidx.at[0, pj]],
                             rowbufs.at[pj], gsem.at[pj])

        def edge_chunk(j, carry):
            jb, b, rg, bg = carry
            p = b & 1
            pg = bg & 1
            slot = j & (NBUF - 1)

            @pl.when(jnp.logical_and(jb == BANK - G, b < NBANKS - 1))
            def _():
                for _k in range(3):
                    pltpu.make_async_copy(
                        dst_hbm.at[pl.ds(base_row, BANK)],
                        didx.at[1 - p], isem).wait()

            pltpu.make_async_copy(table_hbm.at[sidx.at[p, jb]],
                                  rowbufs.at[slot], gsem.at[slot]).wait()

            def scale(g2, c2):
                vv = vbank[p, jb, pl.ds(g2 * 16, 16)]
                for l in range(16):
                    k = g2 * 16 + l
                    v = vv[l]
                    for jj in range(4):
                        sl = pl.ds(jj * 16, 16)
                        rowbufs[slot, k, sl] = rowbufs[slot, k, sl] * v
                return c2
            lax.fori_loop(0, CHUNK // 16, scale, 0, unroll=2)

            pltpu.async_copy(rowbufs.at[slot], acc.at[didx.at[p, jb]],
                             ssem.at[slot], add=True)

            gslot = (j + G) & (NBUF - 1)

            @pl.when(j >= NBUF - G)
            def _():
                pltpu.make_async_copy(rowbufs.at[gslot],
                                      acc.at[didx.at[0, 0]],
                                      ssem.at[gslot]).wait()

            @pl.when(j + G < CPT)
            def _():
                pltpu.async_copy(table_hbm.at[sidx.at[pg, rg]],
                                 rowbufs.at[gslot], gsem.at[gslot])

            @pl.when(jnp.logical_and(jb == 1, b < NBANKS - 1))
            def _():
                off = base_row + (b + 1) * BANK
                pltpu.async_copy(dst_hbm.at[pl.ds(off, BANK)],
                                 didx.at[1 - p], isem)
                pltpu.async_copy(src_hbm.at[pl.ds(off, BANK)],
                                 sidx.at[1 - p], isem)
                pltpu.async_copy(vals_hbm.at[pl.ds(off, BANK)],
                                 vbank.at[1 - p], isem)

            jb = jb + 1
            wrap = jb == BANK
            b = jnp.where(wrap, b + 1, b)
            jb = jnp.where(wrap, 0, jb)
            rg = rg + 1
            wrapg = rg == BANK
            bg = jnp.where(wrapg, bg + 1, bg)
            rg = jnp.where(wrapg, 0, rg)
            return (jb, b, rg, bg)

        lax.fori_loop(0, CPT, edge_chunk,
                      (jnp.int32(0), jnp.int32(0),
                       jnp.int32(G), jnp.int32(0)))

        for _k in range(NBUF - G):
            s = (CPT - (NBUF - G) + _k) & (NBUF - 1)
            pltpu.make_async_copy(rowbufs.at[s], acc.at[didx.at[0, 0]],
                                  ssem.at[s]).wait()

        plsc.subcore_barrier()

        def export(it, c):
            cid = it * 16 + sid

            @pl.when(cid < NZFULL)
            def _():
                sl = pl.ds(cid * ZROWS, ZROWS)
                pltpu.sync_copy(acc.at[sl], out_hbm.at[sl])
            return c
        lax.fori_loop(0, ZITERS, export, 0)

        @pl.when(sid == 0)
        def _():
            sl = pl.ds(NZFULL * ZROWS, ZREM)
            pltpu.sync_copy(acc.at[sl], out_hbm.at[sl])

    @pl.when(core == 0)
    def _():
        run_direction(rows_hbm, cols_hbm, i_hbm, new_u)

    @pl.when(core == 1)
    def _():
        run_direction(cols_hbm, rows_hbm, u_hbm, new_i)


_propagate = functools.partial(
    pl.kernel,
    out_type=(jax.ShapeDtypeStruct((N_NODES, D), jnp.float32),
              jax.ShapeDtypeStruct((N_NODES, D), jnp.float32)),
    mesh=plsc.VectorSubcoreMesh(core_axis_name="c", subcore_axis_name="s"),
    scratch_types=[
        pltpu.VMEM_SHARED((N_NODES, D), jnp.float32),
        pltpu.VMEM((2, BANK, CHUNK), jnp.int32),
        pltpu.VMEM((2, BANK, CHUNK), jnp.int32),
        pltpu.VMEM((2, BANK, CHUNK), jnp.float32),
        pltpu.VMEM((NBUF, CHUNK, D), jnp.float32),
        pltpu.SemaphoreType.DMA((NBUF,)),
        pltpu.SemaphoreType.DMA((NBUF,)),
        pltpu.SemaphoreType.DMA,
    ],
    compiler_params=pltpu.CompilerParams(use_tc_tiling_on_sc=False),
)(_propagate_body)


ROWS_BLK = 1000


def _final_body(u0, u1, u2, u3, i0, i1, i2, i3, us, vs, umap, imap, out):
    out[0, :, :] = (u0[...] + u1[...] + u2[...] + u3[...]) * 0.25
    out[1, :, :] = (i0[...] + i1[...] + i2[...] + i3[...]) * 0.25
    out[2, :, :] = jnp.dot(us[...], umap[...],
                           preferred_element_type=jnp.float32)
    out[3, :, :] = jnp.dot(vs[...], imap[...],
                           preferred_element_type=jnp.float32)


def _finalize(u0, u1, u2, u3, i0, i1, i2, i3, us, vs, umap, imap):
    row_spec = pl.BlockSpec((ROWS_BLK, D), lambda i: (i, 0))
    map_spec = pl.BlockSpec((D, D), lambda i: (0, 0))
    return pl.pallas_call(
        _final_body,
        grid=(N_NODES // ROWS_BLK,),
        in_specs=[row_spec] * 10 + [map_spec] * 2,
        out_specs=pl.BlockSpec((4, ROWS_BLK, D), lambda i: (0, i, 0)),
        out_shape=jax.ShapeDtypeStruct((4, N_NODES, D), jnp.float32),
    )(u0, u1, u2, u3, i0, i1, i2, i3, us, vs, umap, imap)


def kernel(edge_index, edge_vals, user_preference, item_preference,
           user_map, item_map, U_mul_S, V_mul_S):
    rows = edge_index[0].astype(jnp.int32)
    cols = edge_index[1].astype(jnp.int32)
    vals = edge_vals.astype(jnp.float32)

    pad = E_PAD - E
    rows2 = jnp.concatenate(
        [rows, jnp.zeros((pad,), jnp.int32)]).reshape(ROWS2, CHUNK)
    cols2 = jnp.concatenate(
        [cols, jnp.zeros((pad,), jnp.int32)]).reshape(ROWS2, CHUNK)
    vals2 = jnp.concatenate(
        [vals, jnp.zeros((pad,), jnp.float32)]).reshape(ROWS2, CHUNK)

    u0, i0 = user_preference, item_preference
    u1, i1 = _propagate(rows2, cols2, vals2, u0, i0)
    u2, i2 = _propagate(rows2, cols2, vals2, u1, i1)
    u3, i3 = _propagate(rows2, cols2, vals2, u2, i2)

    return _finalize(u0, u1, u2, u3, i0, i1, i2, i3,
                     U_mul_S, V_mul_S, user_map, item_map)


# speedup vs baseline: 11.9745x; 2.1301x over previous
import functools

import jax
import jax.numpy as jnp
from jax import lax
from jax.experimental import pallas as pl
from jax.experimental.pallas import tpu as pltpu
from jax.experimental.pallas import tpu_sc as plsc

N_NODES = 25000
D = 64
E = 800000
CHUNK = 64
CPT = 782
E_PAD = 16 * CPT * CHUNK
ROWS2 = E_PAD // CHUNK
BANK = 23
NBANKS = CPT // BANK
NBUF = 4
G = 2
ZROWS = 64
NZFULL = N_NODES // ZROWS
ZREM = N_NODES - NZFULL * ZROWS
ZITERS = (NZFULL + 15) // 16


def _propagate_body(rows_hbm, cols_hbm, vals_hbm, u_hbm, i_hbm,
                    new_u, new_i, acc, didx, sidx, vbank, rowbufs,
                    gsem, ssem, isem):
    sid = lax.axis_index("s")
    core = lax.axis_index("c")

    def run_direction(dst_hbm, src_hbm, table_hbm, out_hbm):
        base_row = sid * CPT

        def zero_rowbuf(r, c):
            for j in range(4):
                rowbufs[0, r, pl.ds(j * 16, 16)] = jnp.zeros((16,),
                                                             jnp.float32)
            return c
        lax.fori_loop(0, ZROWS, zero_rowbuf, 0)

        def zero_acc(it, c):
            cid = it * 16 + sid

            @pl.when(cid < NZFULL)
            def _():
                pltpu.sync_copy(rowbufs.at[0],
                                acc.at[pl.ds(cid * ZROWS, ZROWS)])
            return c
        lax.fori_loop(0, ZITERS, zero_acc, 0)

        @pl.when(sid == 0)
        def _():
            pltpu.sync_copy(rowbufs.at[0].at[pl.ds(0, ZREM)],
                            acc.at[pl.ds(NZFULL * ZROWS, ZREM)])

        plsc.subcore_barrier()

        pltpu.sync_copy(dst_hbm.at[pl.ds(base_row, BANK)], didx.at[0])
        pltpu.sync_copy(src_hbm.at[pl.ds(base_row, BANK)], sidx.at[0])
        pltpu.sync_copy(vals_hbm.at[pl.ds(base_row, BANK)], vbank.at[0])
        for pj in range(G):
            pltpu.async_copy(table_hbm.at[sidx.at[0, pj]],
                             rowbufs.at[pj], gsem.at[pj])

        def edge_chunk(j, carry):
            jb, b, rg, bg = carry
            p = b & 1
            pg = bg & 1
            slot = j & (NBUF - 1)

            @pl.when(jnp.logical_and(jb == BANK - G, b < NBANKS - 1))
            def _():
                for _k in range(3):
                    pltpu.make_async_copy(
                        dst_hbm.at[pl.ds(base_row, BANK)],
                        didx.at[1 - p], isem).wait()

            pltpu.make_async_copy(table_hbm.at[sidx.at[p, jb]],
                                  rowbufs.at[slot], gsem.at[slot]).wait()

            for g2 in range(CHUNK // 16):
                vv = vbank[p, jb, pl.ds(g2 * 16, 16)]
                for l in range(16):
                    k = g2 * 16 + l
                    v = vv[l]
                    for jj in range(4):
                        sl = pl.ds(jj * 16, 16)
                        rowbufs[slot, k, sl] = rowbufs[slot, k, sl] * v

            pltpu.async_copy(rowbufs.at[slot], acc.at[didx.at[p, jb]],
                             ssem.at[slot], add=True)

            gslot = (j + G) & (NBUF - 1)

            @pl.when(j >= NBUF - G)
            def _():
                pltpu.make_async_copy(rowbufs.at[gslot],
                                      acc.at[didx.at[0, 0]],
                                      ssem.at[gslot]).wait()

            @pl.when(j + G < CPT)
            def _():
                pltpu.async_copy(table_hbm.at[sidx.at[pg, rg]],
                                 rowbufs.at[gslot], gsem.at[gslot])

            @pl.when(jnp.logical_and(jb == 1, b < NBANKS - 1))
            def _():
                off = base_row + (b + 1) * BANK
                pltpu.async_copy(dst_hbm.at[pl.ds(off, BANK)],
                                 didx.at[1 - p], isem)
                pltpu.async_copy(src_hbm.at[pl.ds(off, BANK)],
                                 sidx.at[1 - p], isem)
                pltpu.async_copy(vals_hbm.at[pl.ds(off, BANK)],
                                 vbank.at[1 - p], isem)

            jb = jb + 1
            wrap = jb == BANK
            b = jnp.where(wrap, b + 1, b)
            jb = jnp.where(wrap, 0, jb)
            rg = rg + 1
            wrapg = rg == BANK
            bg = jnp.where(wrapg, bg + 1, bg)
            rg = jnp.where(wrapg, 0, rg)
            return (jb, b, rg, bg)

        lax.fori_loop(0, CPT, edge_chunk,
                      (jnp.int32(0), jnp.int32(0),
                       jnp.int32(G), jnp.int32(0)))

        for _k in range(NBUF - G):
            s = (CPT - (NBUF - G) + _k) & (NBUF - 1)
            pltpu.make_async_copy(rowbufs.at[s], acc.at[didx.at[0, 0]],
                                  ssem.at[s]).wait()

        plsc.subcore_barrier()

        def export(it, c):
            cid = it * 16 + sid

            @pl.when(cid < NZFULL)
            def _():
                sl = pl.ds(cid * ZROWS, ZROWS)
                pltpu.sync_copy(acc.at[sl], out_hbm.at[sl])
            return c
        lax.fori_loop(0, ZITERS, export, 0)

        @pl.when(sid == 0)
        def _():
            sl = pl.ds(NZFULL * ZROWS, ZREM)
            pltpu.sync_copy(acc.at[sl], out_hbm.at[sl])

    @pl.when(core == 0)
    def _():
        run_direction(rows_hbm, cols_hbm, i_hbm, new_u)

    @pl.when(core == 1)
    def _():
        run_direction(cols_hbm, rows_hbm, u_hbm, new_i)


_propagate = functools.partial(
    pl.kernel,
    out_type=(jax.ShapeDtypeStruct((N_NODES, D), jnp.float32),
              jax.ShapeDtypeStruct((N_NODES, D), jnp.float32)),
    mesh=plsc.VectorSubcoreMesh(core_axis_name="c", subcore_axis_name="s"),
    scratch_types=[
        pltpu.VMEM_SHARED((N_NODES, D), jnp.float32),
        pltpu.VMEM((2, BANK, CHUNK), jnp.int32),
        pltpu.VMEM((2, BANK, CHUNK), jnp.int32),
        pltpu.VMEM((2, BANK, CHUNK), jnp.float32),
        pltpu.VMEM((NBUF, CHUNK, D), jnp.float32),
        pltpu.SemaphoreType.DMA((NBUF,)),
        pltpu.SemaphoreType.DMA((NBUF,)),
        pltpu.SemaphoreType.DMA,
    ],
    compiler_params=pltpu.CompilerParams(use_tc_tiling_on_sc=False),
)(_propagate_body)


ROWS_BLK = 1000


def _final_body(u0, u1, u2, u3, i0, i1, i2, i3, us, vs, umap, imap, out):
    out[0, :, :] = (u0[...] + u1[...] + u2[...] + u3[...]) * 0.25
    out[1, :, :] = (i0[...] + i1[...] + i2[...] + i3[...]) * 0.25
    out[2, :, :] = jnp.dot(us[...], umap[...],
                           preferred_element_type=jnp.float32)
    out[3, :, :] = jnp.dot(vs[...], imap[...],
                           preferred_element_type=jnp.float32)


def _finalize(u0, u1, u2, u3, i0, i1, i2, i3, us, vs, umap, imap):
    row_spec = pl.BlockSpec((ROWS_BLK, D), lambda i: (i, 0))
    map_spec = pl.BlockSpec((D, D), lambda i: (0, 0))
    return pl.pallas_call(
        _final_body,
        grid=(N_NODES // ROWS_BLK,),
        in_specs=[row_spec] * 10 + [map_spec] * 2,
        out_specs=pl.BlockSpec((4, ROWS_BLK, D), lambda i: (0, i, 0)),
        out_shape=jax.ShapeDtypeStruct((4, N_NODES, D), jnp.float32),
    )(u0, u1, u2, u3, i0, i1, i2, i3, us, vs, umap, imap)


def kernel(edge_index, edge_vals, user_preference, item_preference,
           user_map, item_map, U_mul_S, V_mul_S):
    rows = edge_index[0].astype(jnp.int32)
    cols = edge_index[1].astype(jnp.int32)
    vals = edge_vals.astype(jnp.float32)

    pad = E_PAD - E
    rows2 = jnp.concatenate(
        [rows, jnp.zeros((pad,), jnp.int32)]).reshape(ROWS2, CHUNK)
    cols2 = jnp.concatenate(
        [cols, jnp.zeros((pad,), jnp.int32)]).reshape(ROWS2, CHUNK)
    vals2 = jnp.concatenate(
        [vals, jnp.zeros((pad,), jnp.float32)]).reshape(ROWS2, CHUNK)

    u0, i0 = user_preference, item_preference
    u1, i1 = _propagate(rows2, cols2, vals2, u0, i0)
    u2, i2 = _propagate(rows2, cols2, vals2, u1, i1)
    u3, i3 = _propagate(rows2, cols2, vals2, u2, i2)

    return _finalize(u0, u1, u2, u3, i0, i1, i2, i3,
                     U_mul_S, V_mul_S, user_map, item_map)


# speedup vs baseline: 13.2829x; 1.1093x over previous
import functools

import jax
import jax.numpy as jnp
from jax import lax
from jax.experimental import pallas as pl
from jax.experimental.pallas import tpu as pltpu
from jax.experimental.pallas import tpu_sc as plsc

N_NODES = 25000
D = 64
E = 800000
CHUNK = 96
CPT = 522
E_PAD = 16 * CPT * CHUNK
ROWS2 = E_PAD // CHUNK
BANK = 9
NBANKS = CPT // BANK
NBUF = 4
G = 2
ZROWS = 96
NZFULL = N_NODES // ZROWS
ZREM = N_NODES - NZFULL * ZROWS
ZITERS = (NZFULL + 15) // 16


def _propagate_body(rows_hbm, cols_hbm, vals_hbm, u_hbm, i_hbm,
                    new_u, new_i, acc, didx, sidx, vbank, rowbufs,
                    gsem, ssem, isem):
    sid = lax.axis_index("s")
    core = lax.axis_index("c")

    def run_direction(dst_hbm, src_hbm, table_hbm, out_hbm):
        base_row = sid * CPT

        def zero_rowbuf(r, c):
            for j in range(4):
                rowbufs[0, r, pl.ds(j * 16, 16)] = jnp.zeros((16,),
                                                             jnp.float32)
            return c
        lax.fori_loop(0, ZROWS, zero_rowbuf, 0)

        def zero_acc(it, c):
            cid = it * 16 + sid

            @pl.when(cid < NZFULL)
            def _():
                pltpu.sync_copy(rowbufs.at[0],
                                acc.at[pl.ds(cid * ZROWS, ZROWS)])
            return c
        lax.fori_loop(0, ZITERS, zero_acc, 0)

        @pl.when(sid == 0)
        def _():
            pltpu.sync_copy(rowbufs.at[0].at[pl.ds(0, ZREM)],
                            acc.at[pl.ds(NZFULL * ZROWS, ZREM)])

        plsc.subcore_barrier()

        pltpu.sync_copy(dst_hbm.at[pl.ds(base_row, BANK)], didx.at[0])
        pltpu.sync_copy(src_hbm.at[pl.ds(base_row, BANK)], sidx.at[0])
        pltpu.sync_copy(vals_hbm.at[pl.ds(base_row, BANK)], vbank.at[0])
        for pj in range(G):
            pltpu.async_copy(table_hbm.at[sidx.at[0, pj]],
                             rowbufs.at[pj], gsem.at[pj])

        def edge_chunk(j, carry):
            jb, b, rg, bg = carry
            p = b & 1
            pg = bg & 1
            slot = j & (NBUF - 1)

            @pl.when(jnp.logical_and(jb == BANK - G, b < NBANKS - 1))
            def _():
                for _k in range(3):
                    pltpu.make_async_copy(
                        dst_hbm.at[pl.ds(base_row, BANK)],
                        didx.at[1 - p], isem).wait()

            pltpu.make_async_copy(table_hbm.at[sidx.at[p, jb]],
                                  rowbufs.at[slot], gsem.at[slot]).wait()

            for g2 in range(CHUNK // 16):
                vv = vbank[p, jb, pl.ds(g2 * 16, 16)]
                for l in range(16):
                    k = g2 * 16 + l
                    v = vv[l]
                    for jj in range(4):
                        sl = pl.ds(jj * 16, 16)
                        rowbufs[slot, k, sl] = rowbufs[slot, k, sl] * v

            pltpu.async_copy(rowbufs.at[slot], acc.at[didx.at[p, jb]],
                             ssem.at[slot], add=True)

            gslot = (j + G) & (NBUF - 1)

            @pl.when(j >= NBUF - G)
            def _():
                pltpu.make_async_copy(rowbufs.at[gslot],
                                      acc.at[didx.at[0, 0]],
                                      ssem.at[gslot]).wait()

            @pl.when(j + G < CPT)
            def _():
                pltpu.async_copy(table_hbm.at[sidx.at[pg, rg]],
                                 rowbufs.at[gslot], gsem.at[gslot])

            @pl.when(jnp.logical_and(jb == 1, b < NBANKS - 1))
            def _():
                off = base_row + (b + 1) * BANK
                pltpu.async_copy(dst_hbm.at[pl.ds(off, BANK)],
                                 didx.at[1 - p], isem)
                pltpu.async_copy(src_hbm.at[pl.ds(off, BANK)],
                                 sidx.at[1 - p], isem)
                pltpu.async_copy(vals_hbm.at[pl.ds(off, BANK)],
                                 vbank.at[1 - p], isem)

            jb = jb + 1
            wrap = jb == BANK
            b = jnp.where(wrap, b + 1, b)
            jb = jnp.where(wrap, 0, jb)
            rg = rg + 1
            wrapg = rg == BANK
            bg = jnp.where(wrapg, bg + 1, bg)
            rg = jnp.where(wrapg, 0, rg)
            return (jb, b, rg, bg)

        lax.fori_loop(0, CPT, edge_chunk,
                      (jnp.int32(0), jnp.int32(0),
                       jnp.int32(G), jnp.int32(0)))

        for _k in range(NBUF - G):
            s = (CPT - (NBUF - G) + _k) & (NBUF - 1)
            pltpu.make_async_copy(rowbufs.at[s], acc.at[didx.at[0, 0]],
                                  ssem.at[s]).wait()

        plsc.subcore_barrier()

        def export(it, c):
            cid = it * 16 + sid

            @pl.when(cid < NZFULL)
            def _():
                sl = pl.ds(cid * ZROWS, ZROWS)
                pltpu.sync_copy(acc.at[sl], out_hbm.at[sl])
            return c
        lax.fori_loop(0, ZITERS, export, 0)

        @pl.when(sid == 0)
        def _():
            sl = pl.ds(NZFULL * ZROWS, ZREM)
            pltpu.sync_copy(acc.at[sl], out_hbm.at[sl])

    @pl.when(core == 0)
    def _():
        run_direction(rows_hbm, cols_hbm, i_hbm, new_u)

    @pl.when(core == 1)
    def _():
        run_direction(cols_hbm, rows_hbm, u_hbm, new_i)


_propagate = functools.partial(
    pl.kernel,
    out_type=(jax.ShapeDtypeStruct((N_NODES, D), jnp.float32),
              jax.ShapeDtypeStruct((N_NODES, D), jnp.float32)),
    mesh=plsc.VectorSubcoreMesh(core_axis_name="c", subcore_axis_name="s"),
    scratch_types=[
        pltpu.VMEM_SHARED((N_NODES, D), jnp.float32),
        pltpu.VMEM((2, BANK, CHUNK), jnp.int32),
        pltpu.VMEM((2, BANK, CHUNK), jnp.int32),
        pltpu.VMEM((2, BANK, CHUNK), jnp.float32),
        pltpu.VMEM((NBUF, CHUNK, D), jnp.float32),
        pltpu.SemaphoreType.DMA((NBUF,)),
        pltpu.SemaphoreType.DMA((NBUF,)),
        pltpu.SemaphoreType.DMA,
    ],
    compiler_params=pltpu.CompilerParams(use_tc_tiling_on_sc=False),
)(_propagate_body)


ROWS_BLK = 1000


def _final_body(u0, u1, u2, u3, i0, i1, i2, i3, us, vs, umap, imap, out):
    out[0, :, :] = (u0[...] + u1[...] + u2[...] + u3[...]) * 0.25
    out[1, :, :] = (i0[...] + i1[...] + i2[...] + i3[...]) * 0.25
    out[2, :, :] = jnp.dot(us[...], umap[...],
                           preferred_element_type=jnp.float32)
    out[3, :, :] = jnp.dot(vs[...], imap[...],
                           preferred_element_type=jnp.float32)


def _finalize(u0, u1, u2, u3, i0, i1, i2, i3, us, vs, umap, imap):
    row_spec = pl.BlockSpec((ROWS_BLK, D), lambda i: (i, 0))
    map_spec = pl.BlockSpec((D, D), lambda i: (0, 0))
    return pl.pallas_call(
        _final_body,
        grid=(N_NODES // ROWS_BLK,),
        in_specs=[row_spec] * 10 + [map_spec] * 2,
        out_specs=pl.BlockSpec((4, ROWS_BLK, D), lambda i: (0, i, 0)),
        out_shape=jax.ShapeDtypeStruct((4, N_NODES, D), jnp.float32),
    )(u0, u1, u2, u3, i0, i1, i2, i3, us, vs, umap, imap)


def kernel(edge_index, edge_vals, user_preference, item_preference,
           user_map, item_map, U_mul_S, V_mul_S):
    rows = edge_index[0].astype(jnp.int32)
    cols = edge_index[1].astype(jnp.int32)
    vals = edge_vals.astype(jnp.float32)

    pad = E_PAD - E
    rows2 = jnp.concatenate(
        [rows, jnp.zeros((pad,), jnp.int32)]).reshape(ROWS2, CHUNK)
    cols2 = jnp.concatenate(
        [cols, jnp.zeros((pad,), jnp.int32)]).reshape(ROWS2, CHUNK)
    vals2 = jnp.concatenate(
        [vals, jnp.zeros((pad,), jnp.float32)]).reshape(ROWS2, CHUNK)

    u0, i0 = user_preference, item_preference
    u1, i1 = _propagate(rows2, cols2, vals2, u0, i0)
    u2, i2 = _propagate(rows2, cols2, vals2, u1, i1)
    u3, i3 = _propagate(rows2, cols2, vals2, u2, i2)

    return _finalize(u0, u1, u2, u3, i0, i1, i2, i3,
                     U_mul_S, V_mul_S, user_map, item_map)
